# baseline (speedup 1.0000x reference)
import functools

import jax
import jax.numpy as jnp
from jax import lax
from jax.experimental import pallas as pl
from jax.experimental.pallas import tpu as pltpu
from jax.experimental.pallas import tpu_sc as plsc

N = 10000
E = 320000
D = 128
G = 50
NPG = 200
NPROT = 10

NC = 2
NS = 16
NW = NC * NS
C = 64
NCH = 158
EPW = NCH * C
EPAD = NW * EPW
NP = 10240
RPS = NP // NS
RCH = 80
NZ = RPS // RCH
DEGW = 16
IDXSHIFT = 14


def _sc_mesh():
    return plsc.VectorSubcoreMesh(
        core_axis_name="c", subcore_axis_name="s",
        num_cores=NC, num_subcores=NS)


def _zero_fill(buf, nrows, width):
    zero16 = jnp.zeros((16,), jnp.float32)

    def _f(i, carry):
        for jj in range(width // 16):
            buf[i, pl.ds(jj * 16, 16)] = zero16
        return carry
    lax.fori_loop(0, nrows, _f, 0)


def _agg_body(x_hbm, packed_hbm, out,
              packv, srcv, dstv, rows, accum, sem0, sem1):
    cid = lax.axis_index("c")
    sid = lax.axis_index("s")
    w = cid * NS + sid
    base = sid * RPS

    _zero_fill(rows.at[0], RCH, D)
    for z in range(NZ):
        pltpu.sync_copy(rows.at[0, pl.ds(0, RCH)],
                        accum.at[pl.ds(base + z * RCH, RCH)])

    pltpu.sync_copy(packed_hbm.at[w], packv)

    def _unpack(j, b):
        for q in range(C // 16):
            v = packv[j, pl.ds(q * 16, 16)]
            srcv[b, pl.ds(q * 16, 16)] = v & ((1 << IDXSHIFT) - 1)
            dstv[b, pl.ds(q * 16, 16)] = v >> IDXSHIFT

    plsc.subcore_barrier()

    _unpack(0, 0)
    pltpu.async_copy(x_hbm.at[srcv.at[0]], rows.at[0], sem0)

    def _pair(t, carry):
        j0 = 2 * t
        _unpack(j0 + 1, 1)
        pltpu.async_copy(x_hbm.at[srcv.at[1]], rows.at[1], sem1)
        pltpu.make_async_copy(x_hbm.at[srcv.at[0]], rows.at[0], sem0).wait()
        pltpu.sync_copy(rows.at[0], accum.at[dstv.at[0]], add=True)
        _unpack(j0 + 2, 0)
        pltpu.async_copy(x_hbm.at[srcv.at[0]], rows.at[0], sem0)
        pltpu.make_async_copy(x_hbm.at[srcv.at[1]], rows.at[1], sem1).wait()
        pltpu.sync_copy(rows.at[1], accum.at[dstv.at[1]], add=True)
        return carry
    lax.fori_loop(0, (NCH - 1) // 2, _pair, 0)

    pltpu.make_async_copy(x_hbm.at[srcv.at[0]], rows.at[0], sem0).wait()
    pltpu.sync_copy(rows.at[0], accum.at[dstv.at[0]], add=True)

    plsc.subcore_barrier()

    sl = pl.ds(base, RPS)
    pltpu.sync_copy(accum.at[sl], out.at[cid, sl])


def _deg_body(packed_hbm, deg, packv, dstv, onesv, stage, degacc, sem0):
    cid = lax.axis_index("c")
    sid = lax.axis_index("s")
    w = cid * NS + sid
    base = sid * RPS

    _zero_fill(stage, RCH, DEGW)
    for z in range(NZ):
        pltpu.sync_copy(stage, degacc.at[pl.ds(base + z * RCH, RCH)])
    ones16 = jnp.ones((16,), jnp.float32)

    def _f(i, carry):
        onesv[i, pl.ds(0, DEGW)] = ones16
        return carry
    lax.fori_loop(0, C, _f, 0)
    pltpu.sync_copy(packed_hbm.at[w], packv)

    plsc.subcore_barrier()

    def _chunk(j, carry):
        for q in range(C // 16):
            dstv[0, pl.ds(q * 16, 16)] = (
                packv[j, pl.ds(q * 16, 16)] >> IDXSHIFT)
        pltpu.sync_copy(onesv, degacc.at[dstv.at[0]], add=True)
        return carry
    lax.fori_loop(0, NCH, _chunk, 0)

    plsc.subcore_barrier()

    sl = pl.ds(base, RPS)
    pltpu.sync_copy(degacc.at[sl], deg.at[cid, sl])


@functools.lru_cache(maxsize=None)
def _make_agg():
    return pl.kernel(
        _agg_body,
        out_type=jax.ShapeDtypeStruct((NC, NP, D), jnp.float32),
        mesh=_sc_mesh(),
        scratch_types=[
            pltpu.VMEM((NCH, C), jnp.int32),
            pltpu.VMEM((2, C), jnp.int32),
            pltpu.VMEM((2, C), jnp.int32),
            pltpu.VMEM((2, C, D), jnp.float32),
            pltpu.VMEM_SHARED((NP, D), jnp.float32),
            pltpu.SemaphoreType.DMA,
            pltpu.SemaphoreType.DMA,
        ],
        compiler_params=pltpu.CompilerParams(use_tc_tiling_on_sc=False),
    )


@functools.lru_cache(maxsize=None)
def _make_deg():
    return pl.kernel(
        _deg_body,
        out_type=jax.ShapeDtypeStruct((NC, NP, DEGW), jnp.float32),
        mesh=_sc_mesh(),
        scratch_types=[
            pltpu.VMEM((NCH, C), jnp.int32),
            pltpu.VMEM((1, C), jnp.int32),
            pltpu.VMEM((C, DEGW), jnp.float32),
            pltpu.VMEM((RCH, DEGW), jnp.float32),
            pltpu.VMEM_SHARED((NP, DEGW), jnp.float32),
            pltpu.SemaphoreType.DMA,
        ],
        compiler_params=pltpu.CompilerParams(use_tc_tiling_on_sc=False),
    )


RB = 2000


def _embed_body(h_ref, w_ref, b_ref, o_ref):
    o_ref[...] = (
        jnp.dot(h_ref[...], w_ref[...], preferred_element_type=jnp.float32)
        + b_ref[...]
    )


def _embed(h, W, b):
    return pl.pallas_call(
        _embed_body,
        grid=(N // RB,),
        in_specs=[
            pl.BlockSpec((RB, D), lambda i: (i, 0)),
            pl.BlockSpec((D, D), lambda i: (0, 0)),
            pl.BlockSpec((1, D), lambda i: (0, 0)),
        ],
        out_specs=pl.BlockSpec((RB, D), lambda i: (i, 0)),
        out_shape=jax.ShapeDtypeStruct((N, D), jnp.float32),
    )(h, W, b.reshape(1, D))


def _sage_update(x, p0, p1, invdeg, wt, wb, b):
    hn = (p0 + p1) * invdeg
    bundle = (
        jnp.dot(x, wt, preferred_element_type=jnp.float32)
        + jnp.dot(hn, wb, preferred_element_type=jnp.float32)
        + b
    )
    nrm = jnp.sqrt(jnp.sum(bundle * bundle, axis=1, keepdims=True))
    return x + jnp.maximum(bundle / jnp.maximum(nrm, 1e-12), 0.0)


def _inv_deg(d0, d1):
    return 1.0 / jnp.maximum(d0[:, :1] + d1[:, :1], 1.0)


def _layer_body(x_ref, p0_ref, p1_ref, d0_ref, d1_ref,
                wt_ref, wb_ref, b_ref, o_ref):
    o_ref[...] = _sage_update(
        x_ref[...], p0_ref[0], p1_ref[0], _inv_deg(d0_ref[0], d1_ref[0]),
        wt_ref[...], wb_ref[...], b_ref[...])


def _layer(x, part, dg, W, b):
    return pl.pallas_call(
        _layer_body,
        grid=(N // RB,),
        in_specs=[
            pl.BlockSpec((RB, D), lambda i: (i, 0)),
            pl.BlockSpec((1, RB, D), lambda i: (0, i, 0)),
            pl.BlockSpec((1, RB, D), lambda i: (1, i, 0)),
            pl.BlockSpec((1, RB, DEGW), lambda i: (0, i, 0)),
            pl.BlockSpec((1, RB, DEGW), lambda i: (1, i, 0)),
            pl.BlockSpec((D, D), lambda i: (0, 0)),
            pl.BlockSpec((D, D), lambda i: (0, 0)),
            pl.BlockSpec((1, D), lambda i: (0, 0)),
        ],
        out_specs=pl.BlockSpec((RB, D), lambda i: (i, 0)),
        out_shape=jax.ShapeDtypeStruct((N, D), jnp.float32),
    )(x, part, part, dg, dg, W[:D], W[D:], b.reshape(1, D))


def _final_body(x_ref, p0_ref, p1_ref, d0_ref, d1_ref, wt_ref, wb_ref, b_ref,
                pp_ref, pn_ref, wfc_ref, o_ref):
    x3 = _sage_update(
        x_ref[...], p0_ref[0], p1_ref[0], _inv_deg(d0_ref[0], d1_ref[0]),
        wt_ref[...], wb_ref[...], b_ref[...])
    wfc = wfc_ref[...]
    pp = pp_ref[...]
    pn = pn_ref[...]
    y = jnp.zeros((), jnp.float32)
    for k in range(NPROT):
        P = pp if k < NPROT // 2 else pn
        diff = x3 - P[k % (NPROT // 2)][None, :]
        d2 = jnp.sum(diff * diff, axis=1)
        sim = jnp.log((d2 + 1.0) / (d2 + 1e-12))
        y = y + jnp.max(sim) * wfc[0, k]
    o_ref[...] = jnp.full((1, 8, D), 1.0 / (1.0 + jnp.exp(-y)), jnp.float32)


def _final(x, part, dg, W, b, p_pos, p_neg, wfc):
    return pl.pallas_call(
        _final_body,
        grid=(G,),
        in_specs=[
            pl.BlockSpec((NPG, D), lambda i: (i, 0)),
            pl.BlockSpec((1, NPG, D), lambda i: (0, i, 0)),
            pl.BlockSpec((1, NPG, D), lambda i: (1, i, 0)),
            pl.BlockSpec((1, NPG, DEGW), lambda i: (0, i, 0)),
            pl.BlockSpec((1, NPG, DEGW), lambda i: (1, i, 0)),
            pl.BlockSpec((D, D), lambda i: (0, 0)),
            pl.BlockSpec((D, D), lambda i: (0, 0)),
            pl.BlockSpec((1, D), lambda i: (0, 0)),
            pl.BlockSpec((NPROT // 2, D), lambda i: (0, 0)),
            pl.BlockSpec((NPROT // 2, D), lambda i: (0, 0)),
            pl.BlockSpec((1, NPROT), lambda i: (0, 0)),
        ],
        out_specs=pl.BlockSpec((1, 8, D), lambda i: (i, 0, 0)),
        out_shape=jax.ShapeDtypeStruct((G, 8, D), jnp.float32),
    )(x, part, part, dg, dg, W[:D], W[D:], b.reshape(1, D), p_pos, p_neg, wfc)


def kernel(h, e, edge_index, graph_ids, W_embed, b_embed,
           W0, b0, W1, b1, W2, b2, p_pos, p_neg, W_fc):
    packed_flat = edge_index[0] | (edge_index[1] << IDXSHIFT)
    pad = jnp.full((EPAD - E,), (NP - 1) << IDXSHIFT, jnp.int32)
    packed = jnp.concatenate([packed_flat, pad]).reshape(NW, NCH, C)

    x0 = _embed(h, W_embed, b_embed)
    dg = _make_deg()(packed)
    p0 = _make_agg()(x0, packed)

    x1 = _layer(x0, p0, dg, W0, b0)
    p1 = _make_agg()(x1, packed)
    x2 = _layer(x1, p1, dg, W1, b1)
    p2 = _make_agg()(x2, packed)

    y = _final(x2, p2, dg, W2, b2, p_pos, p_neg, W_fc)
    return y[:, 0, 0]


# speedup vs baseline: 1.8674x; 1.8674x over previous
import functools

import jax
import jax.numpy as jnp
from jax import lax
from jax.experimental import pallas as pl
from jax.experimental.pallas import tpu as pltpu
from jax.experimental.pallas import tpu_sc as plsc

N = 10000
E = 320000
D = 128
G = 50
NPG = 200
NPROT = 10

NC = 2
NS = 16
NW = NC * NS
C = 64
NCH = 158
EPW = NCH * C
EPAD = NW * EPW
NP = 10240
RPS = NP // NS
RCH = 80
NZ = RPS // RCH
DEGW = 16
IDXSHIFT = 14


def _sc_mesh():
    return plsc.VectorSubcoreMesh(
        core_axis_name="c", subcore_axis_name="s",
        num_cores=NC, num_subcores=NS)


def _zero_fill(buf, nrows, width):
    zero16 = jnp.zeros((16,), jnp.float32)

    def _f(i, carry):
        for jj in range(width // 16):
            buf[i, pl.ds(jj * 16, 16)] = zero16
        return carry
    lax.fori_loop(0, nrows, _f, 0)


def _agg_body(x_hbm, packed_hbm, out,
              packv, srcv, dstv, rows, accum, sem0, sem1):
    cid = lax.axis_index("c")
    sid = lax.axis_index("s")
    w = cid * NS + sid
    base = sid * RPS

    _zero_fill(rows.at[0], RCH, D)
    for z in range(NZ):
        pltpu.sync_copy(rows.at[0, pl.ds(0, RCH)],
                        accum.at[pl.ds(base + z * RCH, RCH)])

    pltpu.sync_copy(packed_hbm.at[w], packv)

    def _unpack(j, b):
        for q in range(C // 16):
            v = packv[j, pl.ds(q * 16, 16)]
            srcv[b, pl.ds(q * 16, 16)] = v & ((1 << IDXSHIFT) - 1)
            dstv[b, pl.ds(q * 16, 16)] = v >> IDXSHIFT

    plsc.subcore_barrier()

    _unpack(0, 0)
    pltpu.async_copy(x_hbm.at[srcv.at[0]], rows.at[0], sem0)

    def _pair(t, carry):
        j0 = 2 * t
        _unpack(j0 + 1, 1)
        pltpu.async_copy(x_hbm.at[srcv.at[1]], rows.at[1], sem1)
        pltpu.make_async_copy(x_hbm.at[srcv.at[0]], rows.at[0], sem0).wait()
        pltpu.sync_copy(rows.at[0], accum.at[dstv.at[0]], add=True)
        _unpack(j0 + 2, 0)
        pltpu.async_copy(x_hbm.at[srcv.at[0]], rows.at[0], sem0)
        pltpu.make_async_copy(x_hbm.at[srcv.at[1]], rows.at[1], sem1).wait()
        pltpu.sync_copy(rows.at[1], accum.at[dstv.at[1]], add=True)
        return carry
    lax.fori_loop(0, (NCH - 1) // 2, _pair, 0)

    pltpu.make_async_copy(x_hbm.at[srcv.at[0]], rows.at[0], sem0).wait()
    pltpu.sync_copy(rows.at[0], accum.at[dstv.at[0]], add=True)

    plsc.subcore_barrier()

    sl = pl.ds(base, RPS)
    pltpu.sync_copy(accum.at[sl], out.at[cid, sl])


def _deg_body(packed_hbm, deg, packv, dstv, onesv, stage, degacc, sem0):
    cid = lax.axis_index("c")
    sid = lax.axis_index("s")
    w = cid * NS + sid
    base = sid * RPS

    _zero_fill(stage, RCH, DEGW)
    for z in range(NZ):
        pltpu.sync_copy(stage, degacc.at[pl.ds(base + z * RCH, RCH)])
    ones16 = jnp.ones((16,), jnp.float32)

    def _f(i, carry):
        onesv[i, pl.ds(0, DEGW)] = ones16
        return carry
    lax.fori_loop(0, C, _f, 0)
    pltpu.sync_copy(packed_hbm.at[w], packv)

    plsc.subcore_barrier()

    def _chunk(j, carry):
        for q in range(C // 16):
            dstv[0, pl.ds(q * 16, 16)] = (
                packv[j, pl.ds(q * 16, 16)] >> IDXSHIFT)
        pltpu.sync_copy(onesv, degacc.at[dstv.at[0]], add=True)
        return carry
    lax.fori_loop(0, NCH, _chunk, 0)

    plsc.subcore_barrier()

    sl = pl.ds(base, RPS)
    pltpu.sync_copy(degacc.at[sl], deg.at[cid, sl])


@functools.lru_cache(maxsize=None)
def _make_agg():
    return pl.kernel(
        _agg_body,
        out_type=jax.ShapeDtypeStruct((NC, NP, D), jnp.float32),
        mesh=_sc_mesh(),
        scratch_types=[
            pltpu.VMEM((NCH, C), jnp.int32),
            pltpu.VMEM((2, C), jnp.int32),
            pltpu.VMEM((2, C), jnp.int32),
            pltpu.VMEM((2, C, D), jnp.float32),
            pltpu.VMEM_SHARED((NP, D), jnp.float32),
            pltpu.SemaphoreType.DMA,
            pltpu.SemaphoreType.DMA,
        ],
        compiler_params=pltpu.CompilerParams(use_tc_tiling_on_sc=False),
    )


@functools.lru_cache(maxsize=None)
def _make_deg():
    return pl.kernel(
        _deg_body,
        out_type=jax.ShapeDtypeStruct((NC, NP, DEGW), jnp.float32),
        mesh=_sc_mesh(),
        scratch_types=[
            pltpu.VMEM((NCH, C), jnp.int32),
            pltpu.VMEM((1, C), jnp.int32),
            pltpu.VMEM((C, DEGW), jnp.float32),
            pltpu.VMEM((RCH, DEGW), jnp.float32),
            pltpu.VMEM_SHARED((NP, DEGW), jnp.float32),
            pltpu.SemaphoreType.DMA,
        ],
        compiler_params=pltpu.CompilerParams(use_tc_tiling_on_sc=False),
    )


RB = 2000


def _embed_body(h_ref, w_ref, b_ref, o_ref):
    o_ref[...] = (
        jnp.dot(h_ref[...], w_ref[...], preferred_element_type=jnp.float32)
        + b_ref[...]
    )


def _embed(h, W, b):
    return pl.pallas_call(
        _embed_body,
        grid=(N // RB,),
        in_specs=[
            pl.BlockSpec((RB, D), lambda i: (i, 0)),
            pl.BlockSpec((D, D), lambda i: (0, 0)),
            pl.BlockSpec((1, D), lambda i: (0, 0)),
        ],
        out_specs=pl.BlockSpec((RB, D), lambda i: (i, 0)),
        out_shape=jax.ShapeDtypeStruct((N, D), jnp.float32),
    )(h, W, b.reshape(1, D))


def _sage_update(x, p0, p1, invdeg, wt, wb, b):
    hn = (p0 + p1) * invdeg
    bundle = (
        jnp.dot(x, wt, preferred_element_type=jnp.float32)
        + jnp.dot(hn, wb, preferred_element_type=jnp.float32)
        + b
    )
    nrm = jnp.sqrt(jnp.sum(bundle * bundle, axis=1, keepdims=True))
    return x + jnp.maximum(bundle / jnp.maximum(nrm, 1e-12), 0.0)


def _inv_deg(d0, d1):
    return 1.0 / jnp.maximum(d0[:, :1] + d1[:, :1], 1.0)


def _layer_body(x_ref, p0_ref, p1_ref, d0_ref, d1_ref,
                wt_ref, wb_ref, b_ref, o_ref):
    o_ref[...] = _sage_update(
        x_ref[...], p0_ref[0], p1_ref[0], _inv_deg(d0_ref[0], d1_ref[0]),
        wt_ref[...], wb_ref[...], b_ref[...])


def _layer(x, part, dg, W, b):
    return pl.pallas_call(
        _layer_body,
        grid=(N // RB,),
        in_specs=[
            pl.BlockSpec((RB, D), lambda i: (i, 0)),
            pl.BlockSpec((1, RB, D), lambda i: (0, i, 0)),
            pl.BlockSpec((1, RB, D), lambda i: (1, i, 0)),
            pl.BlockSpec((1, RB, DEGW), lambda i: (0, i, 0)),
            pl.BlockSpec((1, RB, DEGW), lambda i: (1, i, 0)),
            pl.BlockSpec((D, D), lambda i: (0, 0)),
            pl.BlockSpec((D, D), lambda i: (0, 0)),
            pl.BlockSpec((1, D), lambda i: (0, 0)),
        ],
        out_specs=pl.BlockSpec((RB, D), lambda i: (i, 0)),
        out_shape=jax.ShapeDtypeStruct((N, D), jnp.float32),
    )(x, part, part, dg, dg, W[:D], W[D:], b.reshape(1, D))


def _final_body(x_ref, p0_ref, p1_ref, d0_ref, d1_ref, wt_ref, wb_ref, b_ref,
                pp_ref, pn_ref, wfc_ref, o_ref):
    x3 = _sage_update(
        x_ref[...], p0_ref[0], p1_ref[0], _inv_deg(d0_ref[0], d1_ref[0]),
        wt_ref[...], wb_ref[...], b_ref[...])
    wfc = wfc_ref[...]
    pp = pp_ref[...]
    pn = pn_ref[...]
    y = jnp.zeros((), jnp.float32)
    for k in range(NPROT):
        P = pp if k < NPROT // 2 else pn
        diff = x3 - P[k % (NPROT // 2)][None, :]
        d2 = jnp.sum(diff * diff, axis=1)
        sim = jnp.log((d2 + 1.0) / (d2 + 1e-12))
        y = y + jnp.max(sim) * wfc[0, k]
    o_ref[...] = jnp.full((1, 8, D), 1.0 / (1.0 + jnp.exp(-y)), jnp.float32)


def _final(x, part, dg, W, b, p_pos, p_neg, wfc):
    return pl.pallas_call(
        _final_body,
        grid=(G,),
        in_specs=[
            pl.BlockSpec((NPG, D), lambda i: (i, 0)),
            pl.BlockSpec((1, NPG, D), lambda i: (0, i, 0)),
            pl.BlockSpec((1, NPG, D), lambda i: (1, i, 0)),
            pl.BlockSpec((1, NPG, DEGW), lambda i: (0, i, 0)),
            pl.BlockSpec((1, NPG, DEGW), lambda i: (1, i, 0)),
            pl.BlockSpec((D, D), lambda i: (0, 0)),
            pl.BlockSpec((D, D), lambda i: (0, 0)),
            pl.BlockSpec((1, D), lambda i: (0, 0)),
            pl.BlockSpec((NPROT // 2, D), lambda i: (0, 0)),
            pl.BlockSpec((NPROT // 2, D), lambda i: (0, 0)),
            pl.BlockSpec((1, NPROT), lambda i: (0, 0)),
        ],
        out_specs=pl.BlockSpec((1, 8, D), lambda i: (i, 0, 0)),
        out_shape=jax.ShapeDtypeStruct((G, 8, D), jnp.float32),
    )(x, part, part, dg, dg, W[:D], W[D:], b.reshape(1, D), p_pos, p_neg, wfc)


def kernel(h, e, edge_index, graph_ids, W_embed, b_embed,
           W0, b0, W1, b1, W2, b2, p_pos, p_neg, W_fc):
    packed_flat = edge_index[0] | (edge_index[1] << IDXSHIFT)
    pi = jnp.arange(EPAD - E, dtype=jnp.int32)
    pad = (pi % N) | ((N + pi % (NP - N)) << IDXSHIFT)
    packed = jnp.concatenate([packed_flat, pad]).reshape(NW, NCH, C)

    x0 = _embed(h, W_embed, b_embed)
    dg = _make_deg()(packed)
    p0 = _make_agg()(x0, packed)

    x1 = _layer(x0, p0, dg, W0, b0)
    p1 = _make_agg()(x1, packed)
    x2 = _layer(x1, p1, dg, W1, b1)
    p2 = _make_agg()(x2, packed)

    y = _final(x2, p2, dg, W2, b2, p_pos, p_neg, W_fc)
    return y[:, 0, 0]


# speedup vs baseline: 2.1325x; 1.1420x over previous
import functools

import jax
import jax.numpy as jnp
from jax import lax
from jax.experimental import pallas as pl
from jax.experimental.pallas import tpu as pltpu
from jax.experimental.pallas import tpu_sc as plsc

N = 10000
E = 320000
D = 128
G = 50
NPG = 200
NPROT = 10

NC = 2
NS = 16
NW = NC * NS
C = 64
NCH = 158
EPW = NCH * C
EPAD = NW * EPW
NP = 10240
RPS = NP // NS
RCH = 64
NZ = RPS // RCH
DEGW = 16
IDXSHIFT = 14


def _sc_mesh():
    return plsc.VectorSubcoreMesh(
        core_axis_name="c", subcore_axis_name="s",
        num_cores=NC, num_subcores=NS)


def _zero_fill(buf, nrows, width):
    zero16 = jnp.zeros((16,), jnp.float32)

    def _f(i, carry):
        for jj in range(width // 16):
            buf[i, pl.ds(jj * 16, 16)] = zero16
        return carry
    lax.fori_loop(0, nrows, _f, 0)


def _agg_body(x_hbm, packed_hbm, out, packv, srcv, dstv, rows, accum,
              gsem0, gsem1, gsem2, ssem0, ssem1, ssem2):
    gsem = (gsem0, gsem1, gsem2)
    ssem = (ssem0, ssem1, ssem2)
    cid = lax.axis_index("c")
    sid = lax.axis_index("s")
    w = cid * NS + sid
    base = sid * RPS

    _zero_fill(rows.at[0], RCH, D)
    for z in range(NZ):
        pltpu.sync_copy(rows.at[0], accum.at[pl.ds(base + z * RCH, RCH)])

    pltpu.sync_copy(packed_hbm.at[w], packv)

    def _unpack(j, b):
        for q in range(C // 16):
            v = packv[j, pl.ds(q * 16, 16)]
            srcv[b, pl.ds(q * 16, 16)] = v & ((1 << IDXSHIFT) - 1)
            dstv[b, pl.ds(q * 16, 16)] = v >> IDXSHIFT

    def _gather(j, b):
        _unpack(j, b)
        pltpu.async_copy(x_hbm.at[srcv.at[b]], rows.at[b], gsem[b])

    def _gwait(b):
        pltpu.make_async_copy(x_hbm.at[srcv.at[b]], rows.at[b],
                              gsem[b]).wait()

    def _scatter(b):
        pltpu.async_copy(rows.at[b], accum.at[dstv.at[b]], ssem[b], add=True)

    def _swait(b):
        pltpu.make_async_copy(rows.at[b], accum.at[dstv.at[b]],
                              ssem[b]).wait()

    plsc.subcore_barrier()

    _gather(0, 0)
    _gather(1, 1)
    _gwait(0)
    _scatter(0)
    _gather(2, 2)

    def _tri(t, carry):
        j0 = 3 * t + 1
        for q in range(3):
            b = (1 + q) % 3
            bp = q
            _gwait(b)
            _scatter(b)
            _swait(bp)
            _gather(j0 + q + 2, bp)
        return carry
    lax.fori_loop(0, (NCH - 5) // 3, _tri, 0)

    for j in (NCH - 4, NCH - 3):
        b = j % 3
        bp = (j - 1) % 3
        _gwait(b)
        _scatter(b)
        _swait(bp)
        _gather(j + 2, bp)
    for j in (NCH - 2, NCH - 1):
        b = j % 3
        _gwait(b)
        _scatter(b)
    for j in (NCH - 3, NCH - 2, NCH - 1):
        _swait(j % 3)

    plsc.subcore_barrier()

    sl = pl.ds(base, RPS)
    pltpu.sync_copy(accum.at[sl], out.at[cid, sl])


def _deg_body(packed_hbm, deg, packv, dstv, onesv, stage, degacc, sem0):
    cid = lax.axis_index("c")
    sid = lax.axis_index("s")
    w = cid * NS + sid
    base = sid * RPS

    _zero_fill(stage, RCH, DEGW)
    for z in range(NZ):
        pltpu.sync_copy(stage, degacc.at[pl.ds(base + z * RCH, RCH)])
    ones16 = jnp.ones((16,), jnp.float32)

    def _f(i, carry):
        onesv[i, pl.ds(0, DEGW)] = ones16
        return carry
    lax.fori_loop(0, C, _f, 0)
    pltpu.sync_copy(packed_hbm.at[w], packv)

    plsc.subcore_barrier()

    def _chunk(j, carry):
        for q in range(C // 16):
            dstv[0, pl.ds(q * 16, 16)] = (
                packv[j, pl.ds(q * 16, 16)] >> IDXSHIFT)
        pltpu.sync_copy(onesv, degacc.at[dstv.at[0]], add=True)
        return carry
    lax.fori_loop(0, NCH, _chunk, 0)

    plsc.subcore_barrier()

    sl = pl.ds(base, RPS)
    pltpu.sync_copy(degacc.at[sl], deg.at[cid, sl])


@functools.lru_cache(maxsize=None)
def _make_agg():
    return pl.kernel(
        _agg_body,
        out_type=jax.ShapeDtypeStruct((NC, NP, D), jnp.float32),
        mesh=_sc_mesh(),
        scratch_types=[
            pltpu.VMEM((NCH, C), jnp.int32),
            pltpu.VMEM((3, C), jnp.int32),
            pltpu.VMEM((3, C), jnp.int32),
            pltpu.VMEM((3, C, D), jnp.float32),
            pltpu.VMEM_SHARED((NP, D), jnp.float32),
            pltpu.SemaphoreType.DMA,
            pltpu.SemaphoreType.DMA,
            pltpu.SemaphoreType.DMA,
            pltpu.SemaphoreType.DMA,
            pltpu.SemaphoreType.DMA,
            pltpu.SemaphoreType.DMA,
        ],
        compiler_params=pltpu.CompilerParams(use_tc_tiling_on_sc=False),
    )


@functools.lru_cache(maxsize=None)
def _make_deg():
    return pl.kernel(
        _deg_body,
        out_type=jax.ShapeDtypeStruct((NC, NP, DEGW), jnp.float32),
        mesh=_sc_mesh(),
        scratch_types=[
            pltpu.VMEM((NCH, C), jnp.int32),
            pltpu.VMEM((1, C), jnp.int32),
            pltpu.VMEM((C, DEGW), jnp.float32),
            pltpu.VMEM((RCH, DEGW), jnp.float32),
            pltpu.VMEM_SHARED((NP, DEGW), jnp.float32),
            pltpu.SemaphoreType.DMA,
        ],
        compiler_params=pltpu.CompilerParams(use_tc_tiling_on_sc=False),
    )


RB = 2000


def _embed_body(h_ref, w_ref, b_ref, o_ref):
    o_ref[...] = (
        jnp.dot(h_ref[...], w_ref[...], preferred_element_type=jnp.float32)
        + b_ref[...]
    )


def _embed(h, W, b):
    return pl.pallas_call(
        _embed_body,
        grid=(N // RB,),
        in_specs=[
            pl.BlockSpec((RB, D), lambda i: (i, 0)),
            pl.BlockSpec((D, D), lambda i: (0, 0)),
            pl.BlockSpec((1, D), lambda i: (0, 0)),
        ],
        out_specs=pl.BlockSpec((RB, D), lambda i: (i, 0)),
        out_shape=jax.ShapeDtypeStruct((N, D), jnp.float32),
    )(h, W, b.reshape(1, D))


def _sage_update(x, p0, p1, invdeg, wt, wb, b):
    hn = (p0 + p1) * invdeg
    bundle = (
        jnp.dot(x, wt, preferred_element_type=jnp.float32)
        + jnp.dot(hn, wb, preferred_element_type=jnp.float32)
        + b
    )
    nrm = jnp.sqrt(jnp.sum(bundle * bundle, axis=1, keepdims=True))
    return x + jnp.maximum(bundle / jnp.maximum(nrm, 1e-12), 0.0)


def _inv_deg(d0, d1):
    return 1.0 / jnp.maximum(d0[:, :1] + d1[:, :1], 1.0)


def _layer_body(x_ref, p0_ref, p1_ref, d0_ref, d1_ref,
                wt_ref, wb_ref, b_ref, o_ref):
    o_ref[...] = _sage_update(
        x_ref[...], p0_ref[0], p1_ref[0], _inv_deg(d0_ref[0], d1_ref[0]),
        wt_ref[...], wb_ref[...], b_ref[...])


def _layer(x, part, dg, W, b):
    return pl.pallas_call(
        _layer_body,
        grid=(N // RB,),
        in_specs=[
            pl.BlockSpec((RB, D), lambda i: (i, 0)),
            pl.BlockSpec((1, RB, D), lambda i: (0, i, 0)),
            pl.BlockSpec((1, RB, D), lambda i: (1, i, 0)),
            pl.BlockSpec((1, RB, DEGW), lambda i: (0, i, 0)),
            pl.BlockSpec((1, RB, DEGW), lambda i: (1, i, 0)),
            pl.BlockSpec((D, D), lambda i: (0, 0)),
            pl.BlockSpec((D, D), lambda i: (0, 0)),
            pl.BlockSpec((1, D), lambda i: (0, 0)),
        ],
        out_specs=pl.BlockSpec((RB, D), lambda i: (i, 0)),
        out_shape=jax.ShapeDtypeStruct((N, D), jnp.float32),
    )(x, part, part, dg, dg, W[:D], W[D:], b.reshape(1, D))


def _final_body(x_ref, p0_ref, p1_ref, d0_ref, d1_ref, wt_ref, wb_ref, b_ref,
                pp_ref, pn_ref, wfc_ref, o_ref):
    x3 = _sage_update(
        x_ref[...], p0_ref[0], p1_ref[0], _inv_deg(d0_ref[0], d1_ref[0]),
        wt_ref[...], wb_ref[...], b_ref[...])
    wfc = wfc_ref[...]
    pp = pp_ref[...]
    pn = pn_ref[...]
    y = jnp.zeros((), jnp.float32)
    for k in range(NPROT):
        P = pp if k < NPROT // 2 else pn
        diff = x3 - P[k % (NPROT // 2)][None, :]
        d2 = jnp.sum(diff * diff, axis=1)
        sim = jnp.log((d2 + 1.0) / (d2 + 1e-12))
        y = y + jnp.max(sim) * wfc[0, k]
    o_ref[...] = jnp.full((1, 8, D), 1.0 / (1.0 + jnp.exp(-y)), jnp.float32)


def _final(x, part, dg, W, b, p_pos, p_neg, wfc):
    return pl.pallas_call(
        _final_body,
        grid=(G,),
        in_specs=[
            pl.BlockSpec((NPG, D), lambda i: (i, 0)),
            pl.BlockSpec((1, NPG, D), lambda i: (0, i, 0)),
            pl.BlockSpec((1, NPG, D), lambda i: (1, i, 0)),
            pl.BlockSpec((1, NPG, DEGW), lambda i: (0, i, 0)),
            pl.BlockSpec((1, NPG, DEGW), lambda i: (1, i, 0)),
            pl.BlockSpec((D, D), lambda i: (0, 0)),
            pl.BlockSpec((D, D), lambda i: (0, 0)),
            pl.BlockSpec((1, D), lambda i: (0, 0)),
            pl.BlockSpec((NPROT // 2, D), lambda i: (0, 0)),
            pl.BlockSpec((NPROT // 2, D), lambda i: (0, 0)),
            pl.BlockSpec((1, NPROT), lambda i: (0, 0)),
        ],
        out_specs=pl.BlockSpec((1, 8, D), lambda i: (i, 0, 0)),
        out_shape=jax.ShapeDtypeStruct((G, 8, D), jnp.float32),
    )(x, part, part, dg, dg, W[:D], W[D:], b.reshape(1, D), p_pos, p_neg, wfc)


def kernel(h, e, edge_index, graph_ids, W_embed, b_embed,
           W0, b0, W1, b1, W2, b2, p_pos, p_neg, W_fc):
    packed_flat = edge_index[0] | (edge_index[1] << IDXSHIFT)
    pi = jnp.arange(EPAD - E, dtype=jnp.int32)
    pad = (pi % N) | ((N + pi % (NP - N)) << IDXSHIFT)
    packed = jnp.concatenate([packed_flat, pad]).reshape(NW, NCH, C)

    x0 = _embed(h, W_embed, b_embed)
    dg = _make_deg()(packed)
    p0 = _make_agg()(x0, packed)

    x1 = _layer(x0, p0, dg, W0, b0)
    p1 = _make_agg()(x1, packed)
    x2 = _layer(x1, p1, dg, W1, b1)
    p2 = _make_agg()(x2, packed)

    y = _final(x2, p2, dg, W2, b2, p_pos, p_neg, W_fc)
    return y[:, 0, 0]


# speedup vs baseline: 2.1672x; 1.0163x over previous
import functools

import jax
import jax.numpy as jnp
from jax import lax
from jax.experimental import pallas as pl
from jax.experimental.pallas import tpu as pltpu
from jax.experimental.pallas import tpu_sc as plsc

N = 10000
E = 320000
D = 128
G = 50
NPG = 200
NPROT = 10

NC = 2
NS = 16
NW = NC * NS
C = 64
NCH = 158
EPW = NCH * C
EPAD = NW * EPW
NP = 10240
RPS = NP // NS
RCH = 64
NZ = RPS // RCH
DEGW = 16
IDXSHIFT = 14


def _sc_mesh():
    return plsc.VectorSubcoreMesh(
        core_axis_name="c", subcore_axis_name="s",
        num_cores=NC, num_subcores=NS)


def _zero_fill(buf, nrows, width):
    zero16 = jnp.zeros((16,), jnp.float32)

    def _f(i, carry):
        for jj in range(width // 16):
            buf[i, pl.ds(jj * 16, 16)] = zero16
        return carry
    lax.fori_loop(0, nrows, _f, 0)


def _agg_body(x_hbm, packed_hbm, out, packv, srcv, dstv, rows, accum,
              gsem0, gsem1, gsem2, ssem0, ssem1, ssem2):
    gsem = (gsem0, gsem1, gsem2)
    ssem = (ssem0, ssem1, ssem2)
    cid = lax.axis_index("c")
    sid = lax.axis_index("s")
    w = cid * NS + sid
    base = sid * RPS

    _zero_fill(rows.at[0], RCH, D)
    for z in range(NZ):
        pltpu.sync_copy(rows.at[0], accum.at[pl.ds(base + z * RCH, RCH)])

    pltpu.sync_copy(packed_hbm.at[w], packv)

    def _unpack(j, b):
        for q in range(C // 16):
            v = packv[j, pl.ds(q * 16, 16)]
            srcv[b, pl.ds(q * 16, 16)] = v & ((1 << IDXSHIFT) - 1)
            dstv[b, pl.ds(q * 16, 16)] = v >> IDXSHIFT

    def _gather(j, b):
        _unpack(j, b)
        pltpu.async_copy(x_hbm.at[srcv.at[b]], rows.at[b], gsem[b])

    def _gwait(b):
        pltpu.make_async_copy(x_hbm.at[srcv.at[b]], rows.at[b],
                              gsem[b]).wait()

    def _scatter(b):
        pltpu.async_copy(rows.at[b], accum.at[dstv.at[b]], ssem[b], add=True)

    def _swait(b):
        pltpu.make_async_copy(rows.at[b], accum.at[dstv.at[b]],
                              ssem[b]).wait()

    plsc.subcore_barrier()

    _gather(0, 0)
    _gather(1, 1)
    _gwait(0)
    _scatter(0)
    _gather(2, 2)

    def _tri(t, carry):
        j0 = 3 * t + 1
        for q in range(3):
            b = (1 + q) % 3
            bp = q
            _gwait(b)
            _scatter(b)
            _swait(bp)
            _gather(j0 + q + 2, bp)
        return carry
    lax.fori_loop(0, (NCH - 5) // 3, _tri, 0)

    for j in (NCH - 4, NCH - 3):
        b = j % 3
        bp = (j - 1) % 3
        _gwait(b)
        _scatter(b)
        _swait(bp)
        _gather(j + 2, bp)
    for j in (NCH - 2, NCH - 1):
        b = j % 3
        _gwait(b)
        _scatter(b)
    for j in (NCH - 3, NCH - 2, NCH - 1):
        _swait(j % 3)

    plsc.subcore_barrier()

    sl = pl.ds(base, RPS)
    pltpu.sync_copy(accum.at[sl], out.at[cid, sl])


def _deg_body(packed_hbm, deg, packv, dstv, onesv, stage, degacc, sem0, sem1):
    cid = lax.axis_index("c")
    sid = lax.axis_index("s")
    w = cid * NS + sid
    base = sid * RPS

    _zero_fill(stage, RCH, DEGW)
    for z in range(NZ):
        pltpu.sync_copy(stage, degacc.at[pl.ds(base + z * RCH, RCH)])
    ones16 = jnp.ones((16,), jnp.float32)

    def _f(i, carry):
        onesv[i, pl.ds(0, DEGW)] = ones16
        return carry
    lax.fori_loop(0, C, _f, 0)
    pltpu.sync_copy(packed_hbm.at[w], packv)

    plsc.subcore_barrier()

    def _unp(j, b):
        for q in range(C // 16):
            dstv[b, pl.ds(q * 16, 16)] = (
                packv[j, pl.ds(q * 16, 16)] >> IDXSHIFT)

    sems = (sem0, sem1)

    def _fire(b):
        pltpu.async_copy(onesv, degacc.at[dstv.at[b]], sems[b], add=True)

    def _drain(b):
        pltpu.make_async_copy(onesv, degacc.at[dstv.at[b]], sems[b]).wait()

    _unp(0, 0)
    _fire(0)
    _unp(1, 1)
    _fire(1)

    def _chunk(t, carry):
        j = 2 * t
        _drain(0)
        _unp(j + 2, 0)
        _fire(0)
        _drain(1)
        _unp(j + 3, 1)
        _fire(1)
        return carry
    lax.fori_loop(0, NCH // 2 - 1, _chunk, 0)
    _drain(0)
    _drain(1)

    plsc.subcore_barrier()

    sl = pl.ds(base, RPS)
    pltpu.sync_copy(degacc.at[sl], deg.at[cid, sl])


@functools.lru_cache(maxsize=None)
def _make_agg():
    return pl.kernel(
        _agg_body,
        out_type=jax.ShapeDtypeStruct((NC, NP, D), jnp.float32),
        mesh=_sc_mesh(),
        scratch_types=[
            pltpu.VMEM((NCH, C), jnp.int32),
            pltpu.VMEM((3, C), jnp.int32),
            pltpu.VMEM((3, C), jnp.int32),
            pltpu.VMEM((3, C, D), jnp.float32),
            pltpu.VMEM_SHARED((NP, D), jnp.float32),
            pltpu.SemaphoreType.DMA,
            pltpu.SemaphoreType.DMA,
            pltpu.SemaphoreType.DMA,
            pltpu.SemaphoreType.DMA,
            pltpu.SemaphoreType.DMA,
            pltpu.SemaphoreType.DMA,
        ],
        compiler_params=pltpu.CompilerParams(use_tc_tiling_on_sc=False),
    )


@functools.lru_cache(maxsize=None)
def _make_deg():
    return pl.kernel(
        _deg_body,
        out_type=jax.ShapeDtypeStruct((NC, NP, DEGW), jnp.float32),
        mesh=_sc_mesh(),
        scratch_types=[
            pltpu.VMEM((NCH, C), jnp.int32),
            pltpu.VMEM((2, C), jnp.int32),
            pltpu.VMEM((C, DEGW), jnp.float32),
            pltpu.VMEM((RCH, DEGW), jnp.float32),
            pltpu.VMEM_SHARED((NP, DEGW), jnp.float32),
            pltpu.SemaphoreType.DMA,
            pltpu.SemaphoreType.DMA,
        ],
        compiler_params=pltpu.CompilerParams(use_tc_tiling_on_sc=False),
    )


RB = 2000


def _embed_body(h_ref, w_ref, b_ref, o_ref):
    o_ref[...] = (
        jnp.dot(h_ref[...], w_ref[...], preferred_element_type=jnp.float32)
        + b_ref[...]
    )


def _embed(h, W, b):
    return pl.pallas_call(
        _embed_body,
        grid=(N // RB,),
        in_specs=[
            pl.BlockSpec((RB, D), lambda i: (i, 0)),
            pl.BlockSpec((D, D), lambda i: (0, 0)),
            pl.BlockSpec((1, D), lambda i: (0, 0)),
        ],
        out_specs=pl.BlockSpec((RB, D), lambda i: (i, 0)),
        out_shape=jax.ShapeDtypeStruct((N, D), jnp.float32),
    )(h, W, b.reshape(1, D))


def _sage_update(x, p0, p1, invdeg, wt, wb, b):
    hn = (p0 + p1) * invdeg
    bundle = (
        jnp.dot(x, wt, preferred_element_type=jnp.float32)
        + jnp.dot(hn, wb, preferred_element_type=jnp.float32)
        + b
    )
    nrm = jnp.sqrt(jnp.sum(bundle * bundle, axis=1, keepdims=True))
    return x + jnp.maximum(bundle / jnp.maximum(nrm, 1e-12), 0.0)


def _inv_deg(d0, d1):
    return 1.0 / jnp.maximum(d0[:, :1] + d1[:, :1], 1.0)


def _layer_body(x_ref, p0_ref, p1_ref, d0_ref, d1_ref,
                wt_ref, wb_ref, b_ref, o_ref):
    o_ref[...] = _sage_update(
        x_ref[...], p0_ref[0], p1_ref[0], _inv_deg(d0_ref[0], d1_ref[0]),
        wt_ref[...], wb_ref[...], b_ref[...])


def _layer(x, part, dg, W, b):
    return pl.pallas_call(
        _layer_body,
        grid=(N // RB,),
        in_specs=[
            pl.BlockSpec((RB, D), lambda i: (i, 0)),
            pl.BlockSpec((1, RB, D), lambda i: (0, i, 0)),
            pl.BlockSpec((1, RB, D), lambda i: (1, i, 0)),
            pl.BlockSpec((1, RB, DEGW), lambda i: (0, i, 0)),
            pl.BlockSpec((1, RB, DEGW), lambda i: (1, i, 0)),
            pl.BlockSpec((D, D), lambda i: (0, 0)),
            pl.BlockSpec((D, D), lambda i: (0, 0)),
            pl.BlockSpec((1, D), lambda i: (0, 0)),
        ],
        out_specs=pl.BlockSpec((RB, D), lambda i: (i, 0)),
        out_shape=jax.ShapeDtypeStruct((N, D), jnp.float32),
    )(x, part, part, dg, dg, W[:D], W[D:], b.reshape(1, D))


def _final_body(x_ref, p0_ref, p1_ref, d0_ref, d1_ref, wt_ref, wb_ref, b_ref,
                pp_ref, pn_ref, wfc_ref, o_ref):
    x3 = _sage_update(
        x_ref[...], p0_ref[0], p1_ref[0], _inv_deg(d0_ref[0], d1_ref[0]),
        wt_ref[...], wb_ref[...], b_ref[...])
    wfc = wfc_ref[...]
    pp = pp_ref[...]
    pn = pn_ref[...]
    y = jnp.zeros((), jnp.float32)
    for k in range(NPROT):
        P = pp if k < NPROT // 2 else pn
        diff = x3 - P[k % (NPROT // 2)][None, :]
        d2 = jnp.sum(diff * diff, axis=1)
        sim = jnp.log((d2 + 1.0) / (d2 + 1e-12))
        y = y + jnp.max(sim) * wfc[0, k]
    o_ref[...] = jnp.full((1, 8, D), 1.0 / (1.0 + jnp.exp(-y)), jnp.float32)


def _final(x, part, dg, W, b, p_pos, p_neg, wfc):
    return pl.pallas_call(
        _final_body,
        grid=(G,),
        in_specs=[
            pl.BlockSpec((NPG, D), lambda i: (i, 0)),
            pl.BlockSpec((1, NPG, D), lambda i: (0, i, 0)),
            pl.BlockSpec((1, NPG, D), lambda i: (1, i, 0)),
            pl.BlockSpec((1, NPG, DEGW), lambda i: (0, i, 0)),
            pl.BlockSpec((1, NPG, DEGW), lambda i: (1, i, 0)),
            pl.BlockSpec((D, D), lambda i: (0, 0)),
            pl.BlockSpec((D, D), lambda i: (0, 0)),
            pl.BlockSpec((1, D), lambda i: (0, 0)),
            pl.BlockSpec((NPROT // 2, D), lambda i: (0, 0)),
            pl.BlockSpec((NPROT // 2, D), lambda i: (0, 0)),
            pl.BlockSpec((1, NPROT), lambda i: (0, 0)),
        ],
        out_specs=pl.BlockSpec((1, 8, D), lambda i: (i, 0, 0)),
        out_shape=jax.ShapeDtypeStruct((G, 8, D), jnp.float32),
    )(x, part, part, dg, dg, W[:D], W[D:], b.reshape(1, D), p_pos, p_neg, wfc)


def kernel(h, e, edge_index, graph_ids, W_embed, b_embed,
           W0, b0, W1, b1, W2, b2, p_pos, p_neg, W_fc):
    packed_flat = edge_index[0] | (edge_index[1] << IDXSHIFT)
    pi = jnp.arange(EPAD - E, dtype=jnp.int32)
    pad = (pi % N) | ((N + pi % (NP - N)) << IDXSHIFT)
    packed = jnp.concatenate([packed_flat, pad]).reshape(NW, NCH, C)

    x0 = _embed(h, W_embed, b_embed)
    dg = _make_deg()(packed)
    p0 = _make_agg()(x0, packed)

    x1 = _layer(x0, p0, dg, W0, b0)
    p1 = _make_agg()(x1, packed)
    x2 = _layer(x1, p1, dg, W1, b1)
    p2 = _make_agg()(x2, packed)

    y = _final(x2, p2, dg, W2, b2, p_pos, p_neg, W_fc)
    return y[:, 0, 0]
